# Initial kernel scaffold; baseline (speedup 1.0000x reference)
import jax
import jax.numpy as jnp
from jax.experimental import pallas as pl


def kernel(target, prediction):
    raise NotImplementedError("write your pallas kernel here")


# speedup vs baseline: 1.0409x; 1.0409x over previous
import jax
import jax.numpy as jnp
from jax.experimental import pallas as pl
from jax.experimental.pallas import tpu as pltpu

_B = 128
_V = 100000
_RB = 16
_C = 0.1


def _loss_kernel(targets_ref, x_ref, out_ref):
    b = pl.program_id(0)
    base = b * _RB
    lane_ids = jax.lax.broadcasted_iota(jnp.int32, (1, 128), 1)
    cs = []
    for r in range(_RB):
        t = targets_ref[base + r]
        chunk_start = pl.multiple_of((t // 128) * 128, 128)
        chunk = x_ref[r, pl.ds(chunk_start, 128)].reshape(1, 128)
        lane = t % 128
        cs.append(jnp.sum(jnp.where(lane_ids == lane, chunk, 0.0)))
    correct = jnp.stack(cs).reshape(_RB, 1)
    s = jnp.sum(jnp.maximum(x_ref[...] - correct + _C, 0.0))

    @pl.when(b == 0)
    def _init():
        out_ref[...] = jnp.zeros_like(out_ref)

    out_ref[...] += s

    @pl.when(b == (_B // _RB) - 1)
    def _finish():
        out_ref[...] = (out_ref[...] - _B * _C) / _B


def kernel(target, prediction):
    target = target.astype(jnp.int32)
    out = pl.pallas_call(
        _loss_kernel,
        grid_spec=pltpu.PrefetchScalarGridSpec(
            num_scalar_prefetch=1,
            grid=(_B // _RB,),
            in_specs=[pl.BlockSpec((_RB, _V), lambda i, t: (i, 0))],
            out_specs=pl.BlockSpec((1, 1), lambda i, t: (0, 0)),
        ),
        out_shape=jax.ShapeDtypeStruct((1, 1), jnp.float32),
        compiler_params=pltpu.CompilerParams(
            dimension_semantics=("arbitrary",),
        ),
    )(target, prediction)
    return out.reshape((1,))
